# baseline (speedup 1.0000x reference)
import functools
import math

import jax
import jax.numpy as jnp
from jax import lax
from jax.experimental import pallas as pl
from jax.experimental.pallas import tpu as pltpu
from jax.experimental.pallas import tpu_sc as plsc

B, K, D = 4096, 64, 128
NC, NS = 2, 16
NW = NC * NS
CH = 128

NSLC = 8
BS = B // NSLC
N_MAIN = 2 * BS * K
N_SMALL = 2 * B
N_OUT = B
NCHW = N_MAIN // NW // CH


def _sc_main(ent_hbm, relin_hbm, eids_hbm, rids_hbm, out_t,
             idxe, idxr, e0, e1, r0, r1, o0, o1,
             sge0, sge1, sgr0, sgr1, swb0, swb1):
    c = lax.axis_index("c")
    s = lax.axis_index("s")
    wid = s * NC + c
    base = wid * NCHW * CH

    def add_rows(dst, a, b):
        def row(i, carry):
            for j in range(D // 16):
                sl = pl.ds(j * 16, 16)
                dst[i, sl] = a[i, sl] + b[i, sl]
            return carry
        lax.fori_loop(0, CH, row, 0, unroll=8)

    pltpu.sync_copy(eids_hbm.at[wid], idxe)
    pltpu.sync_copy(rids_hbm.at[wid], idxr)

    pltpu.async_copy(ent_hbm.at[idxe.at[0]], e0, sge0)
    pltpu.async_copy(relin_hbm.at[idxr.at[0]], r0, sgr0)

    def body(i, carry):
        c0 = 2 * i
        c1 = 2 * i + 1
        pltpu.make_async_copy(ent_hbm.at[idxe.at[c0]], e0, sge0).wait()
        pltpu.make_async_copy(relin_hbm.at[idxr.at[c0]], r0, sgr0).wait()
        pltpu.async_copy(ent_hbm.at[idxe.at[c1]], e1, sge1)
        pltpu.async_copy(relin_hbm.at[idxr.at[c1]], r1, sgr1)
        @pl.when(i > 0)
        def _():
            pltpu.make_async_copy(o0, out_t.at[pl.ds(base, CH)], swb0).wait()
        add_rows(o0, e0, r0)
        pltpu.async_copy(o0, out_t.at[pl.ds(base + c0 * CH, CH)], swb0)
        @pl.when(i < NCHW // 2 - 1)
        def _():
            pltpu.async_copy(ent_hbm.at[idxe.at[c0 + 2]], e0, sge0)
            pltpu.async_copy(relin_hbm.at[idxr.at[c0 + 2]], r0, sgr0)
        pltpu.make_async_copy(ent_hbm.at[idxe.at[c1]], e1, sge1).wait()
        pltpu.make_async_copy(relin_hbm.at[idxr.at[c1]], r1, sgr1).wait()
        @pl.when(i > 0)
        def _():
            pltpu.make_async_copy(o1, out_t.at[pl.ds(base, CH)], swb1).wait()
        add_rows(o1, e1, r1)
        pltpu.async_copy(o1, out_t.at[pl.ds(base + c1 * CH, CH)], swb1)
        return carry

    lax.fori_loop(0, NCHW // 2, body, 0)
    pltpu.make_async_copy(o0, out_t.at[pl.ds(base, CH)], swb0).wait()
    pltpu.make_async_copy(o1, out_t.at[pl.ds(base, CH)], swb1).wait()


def _sc_small(ent_hbm, relin_hbm, relout_hbm, sids_hbm, qids_hbm, oids_hbm,
              out_s, out_q, out_o, idx, b0, b1, b2, b3, b4,
              sg0, sg1, sg2, sg3, sg4):
    c = lax.axis_index("c")
    s = lax.axis_index("s")
    wid = s * NC + c
    pltpu.sync_copy(sids_hbm.at[wid], idx.at[pl.ds(0, 2)])
    pltpu.sync_copy(qids_hbm.at[wid], idx.at[pl.ds(2, 2)])
    pltpu.sync_copy(oids_hbm.at[wid], idx.at[pl.ds(4, 1)])
    sbase = wid * 2 * CH
    obase = wid * CH
    pltpu.async_copy(ent_hbm.at[idx.at[0]], b0, sg0)
    pltpu.async_copy(ent_hbm.at[idx.at[1]], b1, sg1)
    pltpu.async_copy(relin_hbm.at[idx.at[2]], b2, sg2)
    pltpu.async_copy(relin_hbm.at[idx.at[3]], b3, sg3)
    pltpu.async_copy(relout_hbm.at[idx.at[4]], b4, sg4)
    pltpu.make_async_copy(ent_hbm.at[idx.at[0]], b0, sg0).wait()
    pltpu.sync_copy(b0, out_s.at[pl.ds(sbase, CH)])
    pltpu.make_async_copy(ent_hbm.at[idx.at[1]], b1, sg1).wait()
    pltpu.sync_copy(b1, out_s.at[pl.ds(sbase + CH, CH)])
    pltpu.make_async_copy(relin_hbm.at[idx.at[2]], b2, sg2).wait()
    pltpu.sync_copy(b2, out_q.at[pl.ds(sbase, CH)])
    pltpu.make_async_copy(relin_hbm.at[idx.at[3]], b3, sg3).wait()
    pltpu.sync_copy(b3, out_q.at[pl.ds(sbase + CH, CH)])
    pltpu.make_async_copy(relout_hbm.at[idx.at[4]], b4, sg4).wait()
    pltpu.sync_copy(b4, out_o.at[pl.ds(obase, CH)])


def _tc_body(t_h, t_t, q_h, q_t, s_h, s_t, ro, w_h, w_t, wq, wk, out, *, bt):
    inv_sqrt_d = 1.0 / math.sqrt(D)

    def encode(t_ref, q_ref, s_ref, w_ref):
        t = t_ref[...]
        kk = jnp.tanh(
            jnp.dot(t.reshape(bt * K, D), wk[...],
                    preferred_element_type=jnp.float32)).reshape(bt, K, D)
        q = jnp.dot(q_ref[...], wq[...], preferred_element_type=jnp.float32)
        logits = jnp.sum(q[:, None, :] * kk, axis=2) * inv_sqrt_d
        m = jnp.max(logits, axis=1, keepdims=True)
        ex = jnp.exp(logits - m)
        alpha_nn = ex / jnp.sum(ex, axis=1, keepdims=True)
        w = w_ref[...]
        w_logic = w / (jnp.sum(w, axis=1, keepdims=True) + 1e-8)
        alpha = 0.5 * (alpha_nn + w_logic)
        agg = jnp.sum(alpha[:, :, None] * t, axis=1)
        return agg + s_ref[...]

    h = encode(t_h, q_h, s_h, w_h)
    t = encode(t_t, q_t, s_t, w_t)
    out[...] = -jnp.sum(jnp.abs(h + ro[...] - t), axis=1)


def kernel(neighbor_head_pos, neighbor_tail_pos, input_relation_ph,
           input_relation_pt, neighbor_weight_ph, neighbor_weight_pt,
           input_triplet_pos, entity_embedding, relation_embedding_out,
           relation_embedding_in, Wq, Wk):
    i32 = jnp.int32
    f32 = jnp.float32
    e_head = neighbor_head_pos[:, :, 1].astype(i32)
    e_tail = neighbor_tail_pos[:, :, 1].astype(i32)
    r_head = neighbor_head_pos[:, :, 0].astype(i32)
    r_tail = neighbor_tail_pos[:, :, 0].astype(i32)
    sids = jnp.concatenate([input_triplet_pos[:, 0], input_triplet_pos[:, 2]]
                           ).astype(i32).reshape(NW, 2, CH)
    qids = jnp.concatenate([input_relation_ph, input_relation_pt]
                           ).astype(i32).reshape(NW, 2, CH)
    oids = input_relation_ph.astype(i32).reshape(NW, 1, CH)

    mesh = plsc.VectorSubcoreMesh(core_axis_name="c", subcore_axis_name="s")

    sc_small = pl.kernel(
        _sc_small,
        out_type=(
            jax.ShapeDtypeStruct((N_SMALL, D), f32),
            jax.ShapeDtypeStruct((N_SMALL, D), f32),
            jax.ShapeDtypeStruct((N_OUT, D), f32),
        ),
        mesh=mesh,
        scratch_types=[pltpu.VMEM((8, CH), i32)]
        + [pltpu.VMEM((CH, D), f32)] * 5
        + [pltpu.SemaphoreType.DMA] * 5,
    )
    g_s, g_q, g_o = sc_small(entity_embedding, relation_embedding_in,
                             relation_embedding_out, sids, qids, oids)

    sc_main = pl.kernel(
        _sc_main,
        out_type=jax.ShapeDtypeStruct((N_MAIN, D), f32),
        mesh=mesh,
        scratch_types=[pltpu.VMEM((NCHW, CH), i32)] * 2
        + [pltpu.VMEM((CH, D), f32)] * 6
        + [pltpu.SemaphoreType.DMA] * 6,
    )

    BT = 128
    NBS = BS // BT
    specw = pl.BlockSpec((BT, K), lambda i: (i, 0))
    specm = pl.BlockSpec((D, D), lambda i: (0, 0))
    tc = pl.pallas_call(
        functools.partial(_tc_body, bt=BT),
        grid=(NBS,),
        in_specs=[pl.BlockSpec((BT, K, D), lambda i: (i, 0, 0)),
                  pl.BlockSpec((BT, K, D), lambda i: (i + NBS, 0, 0)),
                  pl.BlockSpec((BT, D), lambda i: (i, 0)),
                  pl.BlockSpec((BT, D), lambda i: (i + NBS, 0)),
                  pl.BlockSpec((BT, D), lambda i: (i, 0)),
                  pl.BlockSpec((BT, D), lambda i: (i + NBS, 0)),
                  pl.BlockSpec((BT, D), lambda i: (i, 0)),
                  specw, specw, specm, specm],
        out_specs=pl.BlockSpec((BT,), lambda i: (i,)),
        out_shape=jax.ShapeDtypeStruct((BS,), f32),
    )

    wq32 = Wq.astype(f32)
    wk32 = Wk.astype(f32)
    scores = []
    for sl in range(NSLC):
        s0, s1 = sl * BS, (sl + 1) * BS
        eids = jnp.concatenate([e_head[s0:s1].reshape(-1),
                                e_tail[s0:s1].reshape(-1)]
                               ).reshape(NW, NCHW, CH)
        rids = jnp.concatenate([r_head[s0:s1].reshape(-1),
                                r_tail[s0:s1].reshape(-1)]
                               ).reshape(NW, NCHW, CH)
        g_t = sc_main(entity_embedding, relation_embedding_in, eids, rids)
        t3 = g_t.reshape(2 * BS, K, D)
        q2 = jnp.concatenate([g_q[s0:s1], g_q[B + s0:B + s1]])
        s2 = jnp.concatenate([g_s[s0:s1], g_s[B + s0:B + s1]])
        scores.append(tc(t3, t3, q2, q2, s2, s2, g_o[s0:s1],
                         neighbor_weight_ph[s0:s1].astype(f32),
                         neighbor_weight_pt[s0:s1].astype(f32),
                         wq32, wk32))
    return jnp.concatenate(scores)


# speedup vs baseline: 1.0036x; 1.0036x over previous
import functools
import math

import jax
import jax.numpy as jnp
from jax import lax
from jax.experimental import pallas as pl
from jax.experimental.pallas import tpu as pltpu
from jax.experimental.pallas import tpu_sc as plsc

B, K, D = 4096, 64, 128
NC, NS = 2, 16
NW = NC * NS
CH = 128

NSLC = 4
BS = B // NSLC
N_MAIN = 2 * BS * K
N_SMALL = 2 * B
N_OUT = B
NCHW = N_MAIN // NW // CH


def _sc_main(ent_hbm, relin_hbm, eids_hbm, rids_hbm, out_t,
             idxe, idxr, e0, e1, r0, r1, o0, o1,
             sge0, sge1, sgr0, sgr1, swb0, swb1):
    c = lax.axis_index("c")
    s = lax.axis_index("s")
    wid = s * NC + c
    base = wid * NCHW * CH

    def add_rows(dst, a, b):
        def row(i, carry):
            for j in range(D // 16):
                sl = pl.ds(j * 16, 16)
                dst[i, sl] = a[i, sl] + b[i, sl]
            return carry
        lax.fori_loop(0, CH, row, 0, unroll=8)

    pltpu.sync_copy(eids_hbm.at[wid], idxe)
    pltpu.sync_copy(rids_hbm.at[wid], idxr)

    pltpu.async_copy(ent_hbm.at[idxe.at[0]], e0, sge0)
    pltpu.async_copy(relin_hbm.at[idxr.at[0]], r0, sgr0)

    def body(i, carry):
        c0 = 2 * i
        c1 = 2 * i + 1
        pltpu.make_async_copy(ent_hbm.at[idxe.at[c0]], e0, sge0).wait()
        pltpu.make_async_copy(relin_hbm.at[idxr.at[c0]], r0, sgr0).wait()
        pltpu.async_copy(ent_hbm.at[idxe.at[c1]], e1, sge1)
        pltpu.async_copy(relin_hbm.at[idxr.at[c1]], r1, sgr1)
        @pl.when(i > 0)
        def _():
            pltpu.make_async_copy(o0, out_t.at[pl.ds(base, CH)], swb0).wait()
        add_rows(o0, e0, r0)
        pltpu.async_copy(o0, out_t.at[pl.ds(base + c0 * CH, CH)], swb0)
        @pl.when(i < NCHW // 2 - 1)
        def _():
            pltpu.async_copy(ent_hbm.at[idxe.at[c0 + 2]], e0, sge0)
            pltpu.async_copy(relin_hbm.at[idxr.at[c0 + 2]], r0, sgr0)
        pltpu.make_async_copy(ent_hbm.at[idxe.at[c1]], e1, sge1).wait()
        pltpu.make_async_copy(relin_hbm.at[idxr.at[c1]], r1, sgr1).wait()
        @pl.when(i > 0)
        def _():
            pltpu.make_async_copy(o1, out_t.at[pl.ds(base, CH)], swb1).wait()
        add_rows(o1, e1, r1)
        pltpu.async_copy(o1, out_t.at[pl.ds(base + c1 * CH, CH)], swb1)
        return carry

    lax.fori_loop(0, NCHW // 2, body, 0)
    pltpu.make_async_copy(o0, out_t.at[pl.ds(base, CH)], swb0).wait()
    pltpu.make_async_copy(o1, out_t.at[pl.ds(base, CH)], swb1).wait()


def _sc_small(ent_hbm, relin_hbm, relout_hbm, sids_hbm, qids_hbm, oids_hbm,
              out_s, out_q, out_o, idx, b0, b1, b2, b3, b4,
              sg0, sg1, sg2, sg3, sg4):
    c = lax.axis_index("c")
    s = lax.axis_index("s")
    wid = s * NC + c
    pltpu.sync_copy(sids_hbm.at[wid], idx.at[pl.ds(0, 2)])
    pltpu.sync_copy(qids_hbm.at[wid], idx.at[pl.ds(2, 2)])
    pltpu.sync_copy(oids_hbm.at[wid], idx.at[pl.ds(4, 1)])
    sbase = wid * 2 * CH
    obase = wid * CH
    pltpu.async_copy(ent_hbm.at[idx.at[0]], b0, sg0)
    pltpu.async_copy(ent_hbm.at[idx.at[1]], b1, sg1)
    pltpu.async_copy(relin_hbm.at[idx.at[2]], b2, sg2)
    pltpu.async_copy(relin_hbm.at[idx.at[3]], b3, sg3)
    pltpu.async_copy(relout_hbm.at[idx.at[4]], b4, sg4)
    pltpu.make_async_copy(ent_hbm.at[idx.at[0]], b0, sg0).wait()
    pltpu.sync_copy(b0, out_s.at[pl.ds(sbase, CH)])
    pltpu.make_async_copy(ent_hbm.at[idx.at[1]], b1, sg1).wait()
    pltpu.sync_copy(b1, out_s.at[pl.ds(sbase + CH, CH)])
    pltpu.make_async_copy(relin_hbm.at[idx.at[2]], b2, sg2).wait()
    pltpu.sync_copy(b2, out_q.at[pl.ds(sbase, CH)])
    pltpu.make_async_copy(relin_hbm.at[idx.at[3]], b3, sg3).wait()
    pltpu.sync_copy(b3, out_q.at[pl.ds(sbase + CH, CH)])
    pltpu.make_async_copy(relout_hbm.at[idx.at[4]], b4, sg4).wait()
    pltpu.sync_copy(b4, out_o.at[pl.ds(obase, CH)])


def _tc_body(t_h, t_t, q_h, q_t, s_h, s_t, ro, w_h, w_t, wq, wk, out, *, bt):
    inv_sqrt_d = 1.0 / math.sqrt(D)

    def encode(t_ref, q_ref, s_ref, w_ref):
        t = t_ref[...]
        kk = jnp.tanh(
            jnp.dot(t.reshape(bt * K, D), wk[...],
                    preferred_element_type=jnp.float32)).reshape(bt, K, D)
        q = jnp.dot(q_ref[...], wq[...], preferred_element_type=jnp.float32)
        logits = jnp.sum(q[:, None, :] * kk, axis=2) * inv_sqrt_d
        m = jnp.max(logits, axis=1, keepdims=True)
        ex = jnp.exp(logits - m)
        alpha_nn = ex / jnp.sum(ex, axis=1, keepdims=True)
        w = w_ref[...]
        w_logic = w / (jnp.sum(w, axis=1, keepdims=True) + 1e-8)
        alpha = 0.5 * (alpha_nn + w_logic)
        agg = jnp.sum(alpha[:, :, None] * t, axis=1)
        return agg + s_ref[...]

    h = encode(t_h, q_h, s_h, w_h)
    t = encode(t_t, q_t, s_t, w_t)
    out[...] = -jnp.sum(jnp.abs(h + ro[...] - t), axis=1)


def kernel(neighbor_head_pos, neighbor_tail_pos, input_relation_ph,
           input_relation_pt, neighbor_weight_ph, neighbor_weight_pt,
           input_triplet_pos, entity_embedding, relation_embedding_out,
           relation_embedding_in, Wq, Wk):
    i32 = jnp.int32
    f32 = jnp.float32
    e_head = neighbor_head_pos[:, :, 1].astype(i32)
    e_tail = neighbor_tail_pos[:, :, 1].astype(i32)
    r_head = neighbor_head_pos[:, :, 0].astype(i32)
    r_tail = neighbor_tail_pos[:, :, 0].astype(i32)
    sids = jnp.concatenate([input_triplet_pos[:, 0], input_triplet_pos[:, 2]]
                           ).astype(i32).reshape(NW, 2, CH)
    qids = jnp.concatenate([input_relation_ph, input_relation_pt]
                           ).astype(i32).reshape(NW, 2, CH)
    oids = input_relation_ph.astype(i32).reshape(NW, 1, CH)

    mesh = plsc.VectorSubcoreMesh(core_axis_name="c", subcore_axis_name="s")

    sc_small = pl.kernel(
        _sc_small,
        out_type=(
            jax.ShapeDtypeStruct((N_SMALL, D), f32),
            jax.ShapeDtypeStruct((N_SMALL, D), f32),
            jax.ShapeDtypeStruct((N_OUT, D), f32),
        ),
        mesh=mesh,
        scratch_types=[pltpu.VMEM((8, CH), i32)]
        + [pltpu.VMEM((CH, D), f32)] * 5
        + [pltpu.SemaphoreType.DMA] * 5,
    )
    g_s, g_q, g_o = sc_small(entity_embedding, relation_embedding_in,
                             relation_embedding_out, sids, qids, oids)

    sc_main = pl.kernel(
        _sc_main,
        out_type=jax.ShapeDtypeStruct((N_MAIN, D), f32),
        mesh=mesh,
        scratch_types=[pltpu.VMEM((NCHW, CH), i32)] * 2
        + [pltpu.VMEM((CH, D), f32)] * 6
        + [pltpu.SemaphoreType.DMA] * 6,
    )

    BT = 128
    NBS = BS // BT
    specw = pl.BlockSpec((BT, K), lambda i: (i, 0))
    specm = pl.BlockSpec((D, D), lambda i: (0, 0))
    tc = pl.pallas_call(
        functools.partial(_tc_body, bt=BT),
        grid=(NBS,),
        in_specs=[pl.BlockSpec((BT, K, D), lambda i: (i, 0, 0)),
                  pl.BlockSpec((BT, K, D), lambda i: (i + NBS, 0, 0)),
                  pl.BlockSpec((BT, D), lambda i: (i, 0)),
                  pl.BlockSpec((BT, D), lambda i: (i + NBS, 0)),
                  pl.BlockSpec((BT, D), lambda i: (i, 0)),
                  pl.BlockSpec((BT, D), lambda i: (i + NBS, 0)),
                  pl.BlockSpec((BT, D), lambda i: (i, 0)),
                  specw, specw, specm, specm],
        out_specs=pl.BlockSpec((BT,), lambda i: (i,)),
        out_shape=jax.ShapeDtypeStruct((BS,), f32),
    )

    wq32 = Wq.astype(f32)
    wk32 = Wk.astype(f32)
    scores = []
    for sl in range(NSLC):
        s0, s1 = sl * BS, (sl + 1) * BS
        eids = jnp.concatenate([e_head[s0:s1].reshape(-1),
                                e_tail[s0:s1].reshape(-1)]
                               ).reshape(NW, NCHW, CH)
        rids = jnp.concatenate([r_head[s0:s1].reshape(-1),
                                r_tail[s0:s1].reshape(-1)]
                               ).reshape(NW, NCHW, CH)
        g_t = sc_main(entity_embedding, relation_embedding_in, eids, rids)
        t3 = g_t.reshape(2 * BS, K, D)
        q2 = jnp.concatenate([g_q[s0:s1], g_q[B + s0:B + s1]])
        s2 = jnp.concatenate([g_s[s0:s1], g_s[B + s0:B + s1]])
        scores.append(tc(t3, t3, q2, q2, s2, s2, g_o[s0:s1],
                         neighbor_weight_ph[s0:s1].astype(f32),
                         neighbor_weight_pt[s0:s1].astype(f32),
                         wq32, wk32))
    return jnp.concatenate(scores)
